# baseline (speedup 1.0000x reference)
import jax
import jax.numpy as jnp
from jax.experimental import pallas as pl
import jax.experimental.pallas.tpu as pltpu

N = 4096
NFEAT = 128
NHID = 32
NCLASS = 16
TILE = 512
T = N // TILE
NT = 2 * T
NBUF = 3

_DEFAULT = jax.lax.Precision.DEFAULT


def _dot(a, b):
    return jax.lax.dot_general(a, b, (((1,), (0,)), ((), ())),
                               precision=_DEFAULT,
                               preferred_element_type=jnp.float32)


def _gcn_kernel(x_ref, adj_ref, w1_ref, b1_ref, w2_ref, b2_ref, out_ref,
                buf_ref, sem_ref, s1_ref, h_ref, s2_ref):
    k = pl.program_id(0)

    def copy_tile(j):
        return pltpu.make_async_copy(
            adj_ref.at[j // T, pl.ds((j % T) * TILE, TILE), :],
            buf_ref.at[j % NBUF],
            sem_ref.at[j % NBUF])

    @pl.when(k == 0)
    def _():
        copy_tile(0).start()
        copy_tile(1).start()
        s1_ref[...] = _dot(x_ref[...], w1_ref[...])

    @pl.when((k >= 1) & (k + 1 < NT))
    def _():
        copy_tile(k + 1).start()

    @pl.when(k == T + 1)
    def _():
        s2_ref[...] = _dot(h_ref[...], w2_ref[...])

    @pl.when((k >= 1) & (k <= T))
    def _():
        h = _dot(buf_ref[(k - 1) % NBUF], s1_ref[...])
        h_ref[pl.ds((k - 1) * TILE, TILE), :] = jnp.maximum(
            h + b1_ref[...], 0.0)

    @pl.when(k >= T + 1)
    def _():
        o = _dot(buf_ref[(k - 1) % NBUF], s2_ref[...]) + b2_ref[...]
        out_ref[...] = jnp.minimum(o, 0.0) - jnp.log1p(jnp.exp(-jnp.abs(o)))

    @pl.when(k < NT)
    def _():
        copy_tile(k).wait()


@jax.jit
def kernel(x, adj_list, W1, b1, W2, b2):
    grid = (NT + 1,)
    return pl.pallas_call(
        _gcn_kernel,
        grid=grid,
        in_specs=[
            pl.BlockSpec((N, NFEAT), lambda k: (0, 0)),
            pl.BlockSpec(memory_space=pltpu.HBM),
            pl.BlockSpec((NFEAT, NHID), lambda k: (0, 0)),
            pl.BlockSpec((1, NHID), lambda k: (0, 0)),
            pl.BlockSpec((NHID, NCLASS), lambda k: (0, 0)),
            pl.BlockSpec((1, NCLASS), lambda k: (0, 0)),
        ],
        out_specs=pl.BlockSpec(
            (TILE, NCLASS),
            lambda k: (jnp.maximum(k - 1 - T, 0), 0)),
        out_shape=jax.ShapeDtypeStruct((N, NCLASS), jnp.float32),
        scratch_shapes=[
            pltpu.VMEM((NBUF, TILE, N), jnp.float32),
            pltpu.SemaphoreType.DMA((NBUF,)),
            pltpu.VMEM((N, NHID), jnp.float32),
            pltpu.VMEM((N, NHID), jnp.float32),
            pltpu.VMEM((N, NCLASS), jnp.float32),
        ],
    )(x, adj_list, W1, b1.reshape(1, NHID), W2, b2.reshape(1, NCLASS))
